# baseline (speedup 1.0000x reference)
import functools

import jax
import jax.numpy as jnp
from jax import lax
from jax.experimental import pallas as pl
from jax.experimental.pallas import tpu as pltpu
from jax.experimental.pallas import tpu_sc as plsc

N = 100000
D = 128
B = 4096
NC = 2
NS = 16
NW = NC * NS
SPW = B // NW
RB = 512
NG = D // 16
NPAD = 100352
WBLK = 1024


def _bounds_body(seg_ref, out_ref):
    ids = seg_ref[...]
    rows = []
    for w in range(NW):
        lo = jnp.sum((ids < w * SPW).astype(jnp.int32))
        hi = jnp.sum((ids < (w + 1) * SPW).astype(jnp.int32))
        row = jnp.concatenate(
            [jnp.full((1, 1), lo, jnp.int32),
             jnp.full((1, 1), hi, jnp.int32),
             jnp.zeros((1, 14), jnp.int32)], axis=1)
        rows.append(row)
    out_ref[...] = jnp.concatenate(rows, axis=0)


def _compute_bounds(seg_pad):
    out = pl.pallas_call(
        _bounds_body,
        out_shape=jax.ShapeDtypeStruct((NW, 16), jnp.int32),
    )(seg_pad.reshape(NPAD // 128, 128))
    return out.reshape(NW * 16)


def _wt_body(f_ref, ww_ref, bw_ref, out_ref):
    f = f_ref[...].reshape(WBLK // 128, 128, D)
    z = jnp.sum(f * ww_ref[...][0][None, None, :], axis=2)
    out_ref[...] = 1.0 / (1.0 + jnp.exp(-(z + bw_ref[0, 0])))


def _compute_weights(feats, ww_row, bw11):
    return pl.pallas_call(
        _wt_body,
        grid=(NPAD // WBLK,),
        in_specs=[
            pl.BlockSpec((WBLK, D), lambda i: (i, 0)),
            pl.BlockSpec((1, D), lambda i: (0, 0)),
            pl.BlockSpec((1, 1), lambda i: (0, 0),
                         memory_space=pltpu.SMEM),
        ],
        out_specs=pl.BlockSpec((WBLK // 128, 128), lambda i: (i, 0)),
        out_shape=jax.ShapeDtypeStruct((NPAD // 128, 128), jnp.float32),
    )(feats, ww_row, bw11)


def _sc_body(feats_hbm, seg_hbm, w_hbm, bounds_hbm,
             hsum_hbm, hmax_hbm,
             fbuf, sbuf, wbuf, bounds_v, sum_loc, max_loc, max_loc2):
    cid = lax.axis_index("c")
    sid = lax.axis_index("s")
    wid = sid * NC + cid
    base_seg = wid * SPW

    pltpu.sync_copy(bounds_hbm, bounds_v)

    bv = bounds_v[pl.ds(wid * 16, 16)]
    lo = bv[0]
    hi = bv[1]
    g0 = (lo // 16) * 16
    n_tiles = (hi - g0 + RB - 1) // RB

    zero = jnp.zeros((16,), jnp.float32)
    ninf = jnp.full((16,), -jnp.inf, jnp.float32)

    def init_body(i, _):
        for j in range(NG):
            sum_loc[pl.ds(i * D + 16 * j, 16)] = zero
            max_loc[pl.ds(i * D + 16 * j, 16)] = ninf
            max_loc2[pl.ds(i * D + 16 * j, 16)] = ninf
        return 0
    lax.fori_loop(0, SPW + 1, init_body, 0)

    def group_body(st, tstart, gi, _):
        sv = sbuf[pl.ds(16 * gi, 16)]
        wv16 = wbuf[pl.ds(16 * gi, 16)]
        for j16 in range(16):
            r = 16 * gi + j16
            g = st + r
            s = sv[j16]
            ws = wv16[j16]
            inr = (g >= lo) & (g < hi) & (g >= tstart)
            idx = jnp.where(inr, s - base_seg, SPW)
            rows = [fbuf[pl.ds(r * D + 16 * k, 16)] for k in range(NG)]
            ml = max_loc if (j16 % 2 == 0) else max_loc2
            for k in range(NG):
                plsc.addupdate(sum_loc.at[pl.ds(idx * D + 16 * k, 16)],
                               ws * rows[k])
                m = ml[pl.ds(idx * D + 16 * k, 16)]
                ml[pl.ds(idx * D + 16 * k, 16)] = (
                    jnp.maximum(m, rows[k]))
        return 0

    def tile_body(t, _):
        tstart = g0 + t * RB
        st = jnp.minimum(tstart, N - RB)
        st = pl.multiple_of(st, 16)
        pltpu.sync_copy(feats_hbm.at[pl.ds(st * D, RB * D)], fbuf)
        pltpu.sync_copy(seg_hbm.at[pl.ds(st, RB)], sbuf)
        pltpu.sync_copy(w_hbm.at[pl.ds(st, RB)], wbuf)
        return lax.fori_loop(
            0, RB // 16, functools.partial(group_body, st, tstart), 0)

    lax.fori_loop(0, n_tiles, tile_body, 0)

    def merge_body(i, _):
        for j in range(NG):
            a = max_loc[pl.ds(i * D + 16 * j, 16)]
            b = max_loc2[pl.ds(i * D + 16 * j, 16)]
            max_loc[pl.ds(i * D + 16 * j, 16)] = jnp.maximum(a, b)
        return 0
    lax.fori_loop(0, SPW, merge_body, 0)

    pltpu.sync_copy(sum_loc.at[pl.ds(0, SPW * D)],
                    hsum_hbm.at[pl.ds(base_seg * D, SPW * D)])
    pltpu.sync_copy(max_loc.at[pl.ds(0, SPW * D)],
                    hmax_hbm.at[pl.ds(base_seg * D, SPW * D)])


def _segment_readout(feats_flat, seg32, w_flat, bounds):
    mesh = plsc.VectorSubcoreMesh(core_axis_name="c", subcore_axis_name="s")
    f = pl.kernel(
        _sc_body,
        out_type=(jax.ShapeDtypeStruct((B * D,), jnp.float32),
                  jax.ShapeDtypeStruct((B * D,), jnp.float32)),
        mesh=mesh,
        scratch_types=[
            pltpu.VMEM((RB * D,), jnp.float32),
            pltpu.VMEM((RB,), jnp.int32),
            pltpu.VMEM((RB,), jnp.float32),
            pltpu.VMEM((NW * 16,), jnp.int32),
            pltpu.VMEM(((SPW + 1) * D,), jnp.float32),
            pltpu.VMEM(((SPW + 1) * D,), jnp.float32),
            pltpu.VMEM(((SPW + 1) * D,), jnp.float32),
        ],
    )
    return f(feats_flat, seg32, w_flat, bounds)


def _mlp_body(hs_ref, hm_ref, w1a_ref, w1b_ref, b1_ref, g_ref, be_ref,
              w2_ref, b2_ref, out_ref):
    hs = hs_ref[...]
    hm = hm_ref[...]
    hm = jnp.where(jnp.isfinite(hm), hm, 0.0)
    x = (jnp.dot(hs, w1a_ref[...], precision=lax.Precision.HIGHEST)
         + jnp.dot(hm, w1b_ref[...], precision=lax.Precision.HIGHEST)
         + b1_ref[...])
    x = jnp.maximum(x, 0.0)
    mean = jnp.mean(x, axis=0, keepdims=True)
    var = jnp.mean((x - mean) * (x - mean), axis=0, keepdims=True)
    y = (x - mean) * lax.rsqrt(var + 1e-5) * g_ref[...] + be_ref[...]
    out_ref[...] = jnp.dot(y, w2_ref[...],
                           precision=lax.Precision.HIGHEST) + b2_ref[...]


def _mlp(hsum, hmax, w1a, w1b, b1, gamma, beta, w2p, b2p):
    return pl.pallas_call(
        _mlp_body,
        out_shape=jax.ShapeDtypeStruct((B, 128), jnp.float32),
    )(hsum, hmax, w1a, w1b, b1, gamma, beta, w2p, b2p)


@jax.jit
def kernel(feats, segment_ids, Ww, bw, W1, b1, gamma, beta, W2, b2):
    seg32 = segment_ids.astype(jnp.int32)
    seg_pad = jnp.concatenate(
        [seg32, jnp.full((NPAD - N,), jnp.int32(1 << 30), jnp.int32)])
    bounds = _compute_bounds(seg_pad)
    w2d = _compute_weights(feats, Ww.reshape(1, D),
                           bw.reshape(1, 1))
    hsum, hmax = _segment_readout(feats.reshape(N * D), seg32,
                                  w2d.reshape(NPAD), bounds)
    hsum = hsum.reshape(B, D)
    hmax = hmax.reshape(B, D)

    T = W2.shape[0]
    w1a = jnp.transpose(W1[:, :D])
    w1b = jnp.transpose(W1[:, D:])
    w2p = jnp.zeros((128, 128), jnp.float32).at[:, :T].set(jnp.transpose(W2))
    b2p = jnp.zeros((1, 128), jnp.float32).at[0, :T].set(b2)
    out = _mlp(hsum, hmax, w1a, w1b, b1[None, :], gamma[None, :],
               beta[None, :], w2p, b2p)
    return out[:, :T]


# speedup vs baseline: 1.0090x; 1.0090x over previous
import functools

import jax
import jax.numpy as jnp
from jax import lax
from jax.experimental import pallas as pl
from jax.experimental.pallas import tpu as pltpu
from jax.experimental.pallas import tpu_sc as plsc

N = 100000
D = 128
B = 4096
NC = 2
NS = 16
NW = NC * NS
SPW = B // NW
RB = 512
NG = D // 16
NPAD = 100352
WBLK = 1024


def _bounds_body(seg_ref, out_ref):
    ids = seg_ref[...]
    rows = []
    for w in range(NW):
        lo = jnp.sum((ids < w * SPW).astype(jnp.int32))
        hi = jnp.sum((ids < (w + 1) * SPW).astype(jnp.int32))
        row = jnp.concatenate(
            [jnp.full((1, 1), lo, jnp.int32),
             jnp.full((1, 1), hi, jnp.int32),
             jnp.zeros((1, 14), jnp.int32)], axis=1)
        rows.append(row)
    out_ref[...] = jnp.concatenate(rows, axis=0)


def _compute_bounds(seg_pad):
    out = pl.pallas_call(
        _bounds_body,
        out_shape=jax.ShapeDtypeStruct((NW, 16), jnp.int32),
    )(seg_pad.reshape(NPAD // 128, 128))
    return out.reshape(NW * 16)


def _wt_body(f_ref, ww_ref, bw_ref, out_ref):
    f = f_ref[...].reshape(WBLK // 128, 128, D)
    z = jnp.sum(f * ww_ref[...][0][None, None, :], axis=2)
    out_ref[...] = 1.0 / (1.0 + jnp.exp(-(z + bw_ref[0, 0])))


def _compute_weights(feats, ww_row, bw11):
    return pl.pallas_call(
        _wt_body,
        grid=(NPAD // WBLK,),
        in_specs=[
            pl.BlockSpec((WBLK, D), lambda i: (i, 0)),
            pl.BlockSpec((1, D), lambda i: (0, 0)),
            pl.BlockSpec((1, 1), lambda i: (0, 0),
                         memory_space=pltpu.SMEM),
        ],
        out_specs=pl.BlockSpec((WBLK // 128, 128), lambda i: (i, 0)),
        out_shape=jax.ShapeDtypeStruct((NPAD // 128, 128), jnp.float32),
    )(feats, ww_row, bw11)


def _sc_body(feats_hbm, seg_hbm, w_hbm, bounds_hbm,
             hsum_hbm, hmax_hbm,
             fbuf, sbuf, wbuf, bounds_v, sum_loc, max_loc):
    cid = lax.axis_index("c")
    sid = lax.axis_index("s")
    wid = sid * NC + cid
    base_seg = wid * SPW

    pltpu.sync_copy(bounds_hbm, bounds_v)

    bv = bounds_v[pl.ds(wid * 16, 16)]
    lo = bv[0]
    hi = bv[1]
    g0 = (lo // 16) * 16
    n_tiles = (hi - g0 + RB - 1) // RB

    zero = jnp.zeros((16,), jnp.float32)
    ninf = jnp.full((16,), -jnp.inf, jnp.float32)

    def init_body(i, _):
        for j in range(NG):
            sum_loc[pl.ds(i * D + 16 * j, 16)] = zero
            max_loc[pl.ds(i * D + 16 * j, 16)] = ninf
        return 0
    lax.fori_loop(0, SPW + 1, init_body, 0)

    def group_body(st, tstart, gi, _):
        sv = sbuf[pl.ds(16 * gi, 16)]
        wv16 = wbuf[pl.ds(16 * gi, 16)]
        for j16 in range(16):
            r = 16 * gi + j16
            g = st + r
            s = sv[j16]
            ws = wv16[j16]
            inr = (g >= lo) & (g < hi) & (g >= tstart)
            idx = jnp.where(inr, s - base_seg, SPW)
            rows = [fbuf[pl.ds(r * D + 16 * k, 16)] for k in range(NG)]
            for k in range(NG):
                plsc.addupdate(sum_loc.at[pl.ds(idx * D + 16 * k, 16)],
                               ws * rows[k])
                m = max_loc[pl.ds(idx * D + 16 * k, 16)]
                max_loc[pl.ds(idx * D + 16 * k, 16)] = (
                    jnp.maximum(m, rows[k]))
        return 0

    def tile_body(t, _):
        tstart = g0 + t * RB
        st = jnp.minimum(tstart, N - RB)
        st = pl.multiple_of(st, 16)
        pltpu.sync_copy(feats_hbm.at[pl.ds(st * D, RB * D)], fbuf)
        pltpu.sync_copy(seg_hbm.at[pl.ds(st, RB)], sbuf)
        pltpu.sync_copy(w_hbm.at[pl.ds(st, RB)], wbuf)
        return lax.fori_loop(
            0, RB // 16, functools.partial(group_body, st, tstart), 0)

    lax.fori_loop(0, n_tiles, tile_body, 0)

    pltpu.sync_copy(sum_loc.at[pl.ds(0, SPW * D)],
                    hsum_hbm.at[pl.ds(base_seg * D, SPW * D)])
    pltpu.sync_copy(max_loc.at[pl.ds(0, SPW * D)],
                    hmax_hbm.at[pl.ds(base_seg * D, SPW * D)])


def _segment_readout(feats_flat, seg32, w_flat, bounds):
    mesh = plsc.VectorSubcoreMesh(core_axis_name="c", subcore_axis_name="s")
    f = pl.kernel(
        _sc_body,
        out_type=(jax.ShapeDtypeStruct((B * D,), jnp.float32),
                  jax.ShapeDtypeStruct((B * D,), jnp.float32)),
        mesh=mesh,
        scratch_types=[
            pltpu.VMEM((RB * D,), jnp.float32),
            pltpu.VMEM((RB,), jnp.int32),
            pltpu.VMEM((RB,), jnp.float32),
            pltpu.VMEM((NW * 16,), jnp.int32),
            pltpu.VMEM(((SPW + 1) * D,), jnp.float32),
            pltpu.VMEM(((SPW + 1) * D,), jnp.float32),
        ],
    )
    return f(feats_flat, seg32, w_flat, bounds)


def _mlp_body(hs_ref, hm_ref, w1a_ref, w1b_ref, b1_ref, g_ref, be_ref,
              w2_ref, b2_ref, out_ref):
    hs = hs_ref[...]
    hm = hm_ref[...]
    hm = jnp.where(jnp.isfinite(hm), hm, 0.0)
    x = (jnp.dot(hs, w1a_ref[...], precision=lax.Precision.HIGHEST)
         + jnp.dot(hm, w1b_ref[...], precision=lax.Precision.HIGHEST)
         + b1_ref[...])
    x = jnp.maximum(x, 0.0)
    mean = jnp.mean(x, axis=0, keepdims=True)
    var = jnp.mean((x - mean) * (x - mean), axis=0, keepdims=True)
    y = (x - mean) * lax.rsqrt(var + 1e-5) * g_ref[...] + be_ref[...]
    out_ref[...] = jnp.dot(y, w2_ref[...],
                           precision=lax.Precision.HIGHEST) + b2_ref[...]


def _mlp(hsum, hmax, w1a, w1b, b1, gamma, beta, w2p, b2p):
    return pl.pallas_call(
        _mlp_body,
        out_shape=jax.ShapeDtypeStruct((B, 128), jnp.float32),
    )(hsum, hmax, w1a, w1b, b1, gamma, beta, w2p, b2p)


@jax.jit
def kernel(feats, segment_ids, Ww, bw, W1, b1, gamma, beta, W2, b2):
    seg32 = segment_ids.astype(jnp.int32)
    seg_pad = jnp.concatenate(
        [seg32, jnp.full((NPAD - N,), jnp.int32(1 << 30), jnp.int32)])
    bounds = _compute_bounds(seg_pad)
    w2d = _compute_weights(feats, Ww.reshape(1, D),
                           bw.reshape(1, 1))
    hsum, hmax = _segment_readout(feats.reshape(N * D), seg32,
                                  w2d.reshape(NPAD), bounds)
    hsum = hsum.reshape(B, D)
    hmax = hmax.reshape(B, D)

    T = W2.shape[0]
    w1a = jnp.transpose(W1[:, :D])
    w1b = jnp.transpose(W1[:, D:])
    w2p = jnp.zeros((128, 128), jnp.float32).at[:, :T].set(jnp.transpose(W2))
    b2p = jnp.zeros((1, 128), jnp.float32).at[0, :T].set(b2)
    out = _mlp(hsum, hmax, w1a, w1b, b1[None, :], gamma[None, :],
               beta[None, :], w2p, b2p)
    return out[:, :T]
